# baseline (speedup 1.0000x reference)
import jax
import jax.numpy as jnp
from jax import lax
from jax.experimental import pallas as pl
from jax.experimental.pallas import tpu as pltpu
from jax.experimental.pallas import tpu_sc as plsc

NC = 2
NS = 16
L = 16
NW = NC * NS

B = 16384
F = 128
ROWS_PER_W = B // NW
CHUNK = 128
NCHUNK = ROWS_PER_W // CHUNK
GROUPS = CHUNK // L
VIEW_COLS = CHUNK
VROWS_PER_W = ROWS_PER_W // VIEW_COLS


def _mf_body(user_hbm, item_hbm, ut_hbm, it_hbm, w_hbm, b_hbm, out_hbm,
             uidx, iidx, ubuf0, ubuf1, vbuf0, vbuf1, outv, wv, bv, pacc,
             usem0, usem1, vsem0, vsem1):
    wid = lax.axis_index("s") * NC + lax.axis_index("c")
    base = wid * VROWS_PER_W

    pltpu.sync_copy(user_hbm.at[pl.ds(base, VROWS_PER_W)], uidx)
    pltpu.sync_copy(item_hbm.at[pl.ds(base, VROWS_PER_W)], iidx)
    pltpu.sync_copy(w_hbm, wv)
    pltpu.sync_copy(b_hbm, bv)

    ubufs = (ubuf0, ubuf1)
    vbufs = (vbuf0, vbuf1)
    usems = (usem0, usem1)
    vsems = (vsem0, vsem1)

    def gather(j, slot):
        pltpu.make_async_copy(ut_hbm.at[uidx.at[j]], ubufs[slot], usems[slot]).start()
        pltpu.make_async_copy(it_hbm.at[iidx.at[j]], vbufs[slot], vsems[slot]).start()

    def wait(j, slot):
        pltpu.make_async_copy(ut_hbm.at[uidx.at[j]], ubufs[slot], usems[slot]).wait()
        pltpu.make_async_copy(it_hbm.at[iidx.at[j]], vbufs[slot], vsems[slot]).wait()

    lane = lax.broadcasted_iota(jnp.int32, (L,), 0)
    bias = bv[...]
    wregs = [wv[pl.ds(c * L, L)] for c in range(F // L)]

    gather(0, 0)
    for j in range(NCHUNK):
        slot = j % 2
        if j + 1 < NCHUNK:
            gather(j + 1, 1 - slot)
        wait(j, slot)
        ub = ubufs[slot]
        vb = vbufs[slot]

        @plsc.parallel_loop(0, CHUNK, unroll=8)
        def r_body(r):
            acc = ub[r, pl.ds(0, L)] * vb[r, pl.ds(0, L)] * wregs[0]
            for c in range(1, F // L):
                cu = ub[r, pl.ds(c * L, L)]
                cv = vb[r, pl.ds(c * L, L)]
                acc = acc + cu * cv * wregs[c]
            pacc[r, pl.ds(0, L)] = acc

        jrow = j

        @plsc.parallel_loop(0, GROUPS, unroll=4)
        def g_body(g):
            rows = lane + g * L
            cols = [plsc.load_gather(pacc, [rows, jnp.full((L,), l, dtype=jnp.int32)])
                    for l in range(L)]
            while len(cols) > 1:
                cols = [cols[i] + cols[i + 1] for i in range(0, len(cols), 2)]
            outv[jrow, pl.ds(g * L, L)] = cols[0] + bias

    pltpu.sync_copy(outv, out_hbm.at[pl.ds(base, VROWS_PER_W)])


@jax.jit
def _mf(user2d, item2d, user_table, item_table, w_flat, b16):
    kern = pl.kernel(
        _mf_body,
        out_type=jax.ShapeDtypeStruct((B // VIEW_COLS, VIEW_COLS), jnp.float32),
        mesh=plsc.VectorSubcoreMesh(
            core_axis_name="c", subcore_axis_name="s",
            num_cores=NC, num_subcores=NS),
        scratch_types=[
            pltpu.VMEM((VROWS_PER_W, VIEW_COLS), jnp.int32),
            pltpu.VMEM((VROWS_PER_W, VIEW_COLS), jnp.int32),
            pltpu.VMEM((CHUNK, F), jnp.float32),
            pltpu.VMEM((CHUNK, F), jnp.float32),
            pltpu.VMEM((CHUNK, F), jnp.float32),
            pltpu.VMEM((CHUNK, F), jnp.float32),
            pltpu.VMEM((VROWS_PER_W, VIEW_COLS), jnp.float32),
            pltpu.VMEM((F,), jnp.float32),
            pltpu.VMEM((L,), jnp.float32),
            pltpu.VMEM((CHUNK, L), jnp.float32),
            pltpu.SemaphoreType.DMA,
            pltpu.SemaphoreType.DMA,
            pltpu.SemaphoreType.DMA,
            pltpu.SemaphoreType.DMA,
        ],
        compiler_params=pltpu.CompilerParams(needs_layout_passes=False),
    )
    return kern(user2d, item2d, user_table, item_table, w_flat, b16)


def kernel(user, item, user_table, item_table, W, b):
    user2d = user.astype(jnp.int32).reshape(B // VIEW_COLS, VIEW_COLS)
    item2d = item.astype(jnp.int32).reshape(B // VIEW_COLS, VIEW_COLS)
    w_flat = W.reshape(F)
    b16 = jnp.broadcast_to(b.astype(jnp.float32), (L,))
    out = _mf(user2d, item2d, user_table, item_table, w_flat, b16)
    return out.reshape(-1)


# speedup vs baseline: 1.0667x; 1.0667x over previous
import jax
import jax.numpy as jnp
from jax import lax
from jax.experimental import pallas as pl
from jax.experimental.pallas import tpu as pltpu
from jax.experimental.pallas import tpu_sc as plsc

NC = 2
NS = 16
L = 16
NW = NC * NS

B = 16384
F = 128
ROWS_PER_W = B // NW
CHUNK = 128
NCHUNK = ROWS_PER_W // CHUNK
GROUPS = CHUNK // L
VIEW_COLS = CHUNK
VROWS_PER_W = ROWS_PER_W // VIEW_COLS


def _mf_body(user_hbm, item_hbm, ut_hbm, it_hbm, w_hbm, b_hbm, out_hbm,
             uidx, iidx, ubuf0, ubuf1, vbuf0, vbuf1, outv, wv, bv, pacc,
             usem0, usem1, vsem0, vsem1):
    wid = lax.axis_index("s") * NC + lax.axis_index("c")
    base = wid * VROWS_PER_W

    pltpu.sync_copy(user_hbm.at[pl.ds(base, VROWS_PER_W)], uidx)
    pltpu.sync_copy(item_hbm.at[pl.ds(base, VROWS_PER_W)], iidx)
    pltpu.sync_copy(w_hbm, wv)
    pltpu.sync_copy(b_hbm, bv)

    ubufs = (ubuf0, ubuf1)
    vbufs = (vbuf0, vbuf1)
    usems = (usem0, usem1)
    vsems = (vsem0, vsem1)

    def gather(j, slot):
        pltpu.make_async_copy(ut_hbm.at[uidx.at[j]], ubufs[slot], usems[slot]).start()
        pltpu.make_async_copy(it_hbm.at[iidx.at[j]], vbufs[slot], vsems[slot]).start()

    def wait(j, slot):
        pltpu.make_async_copy(ut_hbm.at[uidx.at[j]], ubufs[slot], usems[slot]).wait()
        pltpu.make_async_copy(it_hbm.at[iidx.at[j]], vbufs[slot], vsems[slot]).wait()

    lane = lax.broadcasted_iota(jnp.int32, (L,), 0)
    bias = bv[...]
    wregs = [wv[pl.ds(c * L, L)] for c in range(F // L)]

    gather(0, 0)
    for j in range(NCHUNK):
        slot = j % 2
        if j + 1 < NCHUNK:
            gather(j + 1, 1 - slot)
        wait(j, slot)
        ub = ubufs[slot]
        vb = vbufs[slot]

        @plsc.parallel_loop(0, CHUNK, unroll=2)
        def r_body(r):
            acc = ub[r, pl.ds(0, L)] * vb[r, pl.ds(0, L)] * wregs[0]
            for c in range(1, F // L):
                cu = ub[r, pl.ds(c * L, L)]
                cv = vb[r, pl.ds(c * L, L)]
                acc = acc + cu * cv * wregs[c]
            pacc[r, pl.ds(0, L)] = acc

        jrow = j

        @plsc.parallel_loop(0, GROUPS, unroll=1)
        def g_body(g):
            rows = lane + g * L
            cols = [plsc.load_gather(pacc, [rows, jnp.full((L,), l, dtype=jnp.int32)])
                    for l in range(L)]
            while len(cols) > 1:
                cols = [cols[i] + cols[i + 1] for i in range(0, len(cols), 2)]
            outv[jrow, pl.ds(g * L, L)] = cols[0] + bias

    pltpu.sync_copy(outv, out_hbm.at[pl.ds(base, VROWS_PER_W)])


@jax.jit
def _mf(user2d, item2d, user_table, item_table, w_flat, b16):
    kern = pl.kernel(
        _mf_body,
        out_type=jax.ShapeDtypeStruct((B // VIEW_COLS, VIEW_COLS), jnp.float32),
        mesh=plsc.VectorSubcoreMesh(
            core_axis_name="c", subcore_axis_name="s",
            num_cores=NC, num_subcores=NS),
        scratch_types=[
            pltpu.VMEM((VROWS_PER_W, VIEW_COLS), jnp.int32),
            pltpu.VMEM((VROWS_PER_W, VIEW_COLS), jnp.int32),
            pltpu.VMEM((CHUNK, F), jnp.float32),
            pltpu.VMEM((CHUNK, F), jnp.float32),
            pltpu.VMEM((CHUNK, F), jnp.float32),
            pltpu.VMEM((CHUNK, F), jnp.float32),
            pltpu.VMEM((VROWS_PER_W, VIEW_COLS), jnp.float32),
            pltpu.VMEM((F,), jnp.float32),
            pltpu.VMEM((L,), jnp.float32),
            pltpu.VMEM((CHUNK, L), jnp.float32),
            pltpu.SemaphoreType.DMA,
            pltpu.SemaphoreType.DMA,
            pltpu.SemaphoreType.DMA,
            pltpu.SemaphoreType.DMA,
        ],
        compiler_params=pltpu.CompilerParams(needs_layout_passes=False),
    )
    return kern(user2d, item2d, user_table, item_table, w_flat, b16)


def kernel(user, item, user_table, item_table, W, b):
    user2d = user.astype(jnp.int32).reshape(B // VIEW_COLS, VIEW_COLS)
    item2d = item.astype(jnp.int32).reshape(B // VIEW_COLS, VIEW_COLS)
    w_flat = W.reshape(F)
    b16 = jnp.broadcast_to(b.astype(jnp.float32), (L,))
    out = _mf(user2d, item2d, user_table, item_table, w_flat, b16)
    return out.reshape(-1)


# speedup vs baseline: 1.0687x; 1.0018x over previous
import jax
import jax.numpy as jnp
from jax import lax
from jax.experimental import pallas as pl
from jax.experimental.pallas import tpu as pltpu
from jax.experimental.pallas import tpu_sc as plsc

NC = 2
NS = 16
L = 16
NW = NC * NS

B = 16384
F = 128
ROWS_PER_W = B // NW
CHUNK = 128
NCHUNK = ROWS_PER_W // CHUNK
GROUPS = CHUNK // L
VIEW_COLS = CHUNK
VROWS_PER_W = ROWS_PER_W // VIEW_COLS


def _mf_body(user_hbm, item_hbm, ut_hbm, it_hbm, w_hbm, b_hbm, out_hbm,
             uidx, iidx, ubuf0, ubuf1, vbuf0, vbuf1, outv, wv, bv, pacc,
             usem0, usem1, vsem0, vsem1):
    wid = lax.axis_index("s") * NC + lax.axis_index("c")
    base = wid * VROWS_PER_W

    pltpu.sync_copy(user_hbm.at[pl.ds(base, VROWS_PER_W)], uidx)
    pltpu.sync_copy(item_hbm.at[pl.ds(base, VROWS_PER_W)], iidx)
    pltpu.sync_copy(w_hbm, wv)
    pltpu.sync_copy(b_hbm, bv)

    ubufs = (ubuf0, ubuf1)
    vbufs = (vbuf0, vbuf1)
    usems = (usem0, usem1)
    vsems = (vsem0, vsem1)

    def gather(j, slot):
        pltpu.make_async_copy(ut_hbm.at[uidx.at[j]], ubufs[slot], usems[slot]).start()
        pltpu.make_async_copy(it_hbm.at[iidx.at[j]], vbufs[slot], vsems[slot]).start()

    def wait(j, slot):
        pltpu.make_async_copy(ut_hbm.at[uidx.at[j]], ubufs[slot], usems[slot]).wait()
        pltpu.make_async_copy(it_hbm.at[iidx.at[j]], vbufs[slot], vsems[slot]).wait()

    lane = lax.broadcasted_iota(jnp.int32, (L,), 0)
    bias = bv[...]
    wregs = [wv[pl.ds(c * L, L)] for c in range(F // L)]

    gather(0, 0)
    for j in range(NCHUNK):
        slot = j % 2
        if j + 1 < NCHUNK:
            gather(j + 1, 1 - slot)
        wait(j, slot)
        ub = ubufs[slot]
        vb = vbufs[slot]

        @plsc.parallel_loop(0, CHUNK, unroll=1)
        def r_body(r):
            acc = ub[r, pl.ds(0, L)] * vb[r, pl.ds(0, L)] * wregs[0]
            for c in range(1, F // L):
                cu = ub[r, pl.ds(c * L, L)]
                cv = vb[r, pl.ds(c * L, L)]
                acc = acc + cu * cv * wregs[c]
            pacc[r, pl.ds(0, L)] = acc

        jrow = j

        @plsc.parallel_loop(0, GROUPS, unroll=1)
        def g_body(g):
            rows = lane + g * L
            cols = [plsc.load_gather(pacc, [rows, jnp.full((L,), l, dtype=jnp.int32)])
                    for l in range(L)]
            while len(cols) > 1:
                cols = [cols[i] + cols[i + 1] for i in range(0, len(cols), 2)]
            outv[jrow, pl.ds(g * L, L)] = cols[0] + bias

    pltpu.sync_copy(outv, out_hbm.at[pl.ds(base, VROWS_PER_W)])


@jax.jit
def _mf(user2d, item2d, user_table, item_table, w_flat, b16):
    kern = pl.kernel(
        _mf_body,
        out_type=jax.ShapeDtypeStruct((B // VIEW_COLS, VIEW_COLS), jnp.float32),
        mesh=plsc.VectorSubcoreMesh(
            core_axis_name="c", subcore_axis_name="s",
            num_cores=NC, num_subcores=NS),
        scratch_types=[
            pltpu.VMEM((VROWS_PER_W, VIEW_COLS), jnp.int32),
            pltpu.VMEM((VROWS_PER_W, VIEW_COLS), jnp.int32),
            pltpu.VMEM((CHUNK, F), jnp.float32),
            pltpu.VMEM((CHUNK, F), jnp.float32),
            pltpu.VMEM((CHUNK, F), jnp.float32),
            pltpu.VMEM((CHUNK, F), jnp.float32),
            pltpu.VMEM((VROWS_PER_W, VIEW_COLS), jnp.float32),
            pltpu.VMEM((F,), jnp.float32),
            pltpu.VMEM((L,), jnp.float32),
            pltpu.VMEM((CHUNK, L), jnp.float32),
            pltpu.SemaphoreType.DMA,
            pltpu.SemaphoreType.DMA,
            pltpu.SemaphoreType.DMA,
            pltpu.SemaphoreType.DMA,
        ],
        compiler_params=pltpu.CompilerParams(needs_layout_passes=False),
    )
    return kern(user2d, item2d, user_table, item_table, w_flat, b16)


def kernel(user, item, user_table, item_table, W, b):
    user2d = user.astype(jnp.int32).reshape(B // VIEW_COLS, VIEW_COLS)
    item2d = item.astype(jnp.int32).reshape(B // VIEW_COLS, VIEW_COLS)
    w_flat = W.reshape(F)
    b16 = jnp.broadcast_to(b.astype(jnp.float32), (L,))
    out = _mf(user2d, item2d, user_table, item_table, w_flat, b16)
    return out.reshape(-1)
